# baseline (speedup 1.0000x reference)
import functools

import jax
import jax.numpy as jnp
from jax import lax
from jax.experimental import pallas as pl
from jax.experimental.pallas import tpu as pltpu

D_REPR = 64
TOP_K = 3
THRESH = 0.95
MEM = 5000
T = 200
BT = 256

_NEG_INF = float("-inf")


def _fused_kernel(q_ref, w_ref, gbb_ref, mk_ref, mv_ref,
                  ts_ref, rv_ref, bm_ref, mkn_ref):
    i = pl.program_id(0)

    @pl.when(i == 0)
    def _():
        mk = mk_ref[...]
        mkn_ref[...] = (mk / jnp.maximum(
            jnp.sqrt(jnp.sum(mk * mk, axis=1, keepdims=True)),
            jnp.float32(1e-12))).astype(jnp.bfloat16)

    xt = jnp.swapaxes(q_ref[...], 1, 2)
    inv_d = jnp.float32(1.0 / D_REPR)
    ae = jnp.sum(xt[:, :64, :], axis=1) * inv_d
    ao = jnp.sum(xt[:, 64:, :], axis=1) * inv_d

    mean = (jnp.sum(ae, axis=1, keepdims=True)
            + jnp.sum(ao, axis=1, keepdims=True)) * jnp.float32(1.0 / T)
    dev2 = (jnp.sum((ae - mean) ** 2, axis=1, keepdims=True)
            + jnp.sum((ao - mean) ** 2, axis=1, keepdims=True))
    std = jnp.sqrt(dev2 * jnp.float32(1.0 / (T - 1)))
    mx = jnp.maximum(jnp.max(ae, axis=1, keepdims=True),
                     jnp.max(ao, axis=1, keepdims=True))
    mn = jnp.minimum(jnp.min(ae, axis=1, keepdims=True),
                     jnp.min(ao, axis=1, keepdims=True))
    trend = ao[:, T // 2 - 1:] - ae[:, :1]
    stats = jnp.concatenate([mean, std, mx, mn, trend], axis=1)

    h = lax.dot_general(stats.astype(jnp.bfloat16),
                        w_ref[...].astype(jnp.bfloat16),
                        (((1,), (0,)), ((), ())),
                        preferred_element_type=jnp.float32)
    h = h + gbb_ref[0:1, :]
    mu = jnp.mean(h, axis=1, keepdims=True)
    var = jnp.mean((h - mu) ** 2, axis=1, keepdims=True)
    h = (h - mu) * lax.rsqrt(var + jnp.float32(1e-5))
    h = h * gbb_ref[1:2, :] + gbb_ref[2:3, :]

    qn = h / jnp.maximum(jnp.sqrt(jnp.sum(h * h, axis=1, keepdims=True)),
                         jnp.float32(1e-12))

    sim = lax.dot_general(qn.astype(jnp.bfloat16), mkn_ref[...],
                          (((1,), (1,)), ((), ())),
                          preferred_element_type=jnp.float32)

    vals_col = mv_ref[...]
    iota = lax.broadcasted_iota(jnp.int32, (BT, MEM), 1)
    work = sim
    top_s = []
    top_v = []
    for _ in range(TOP_K):
        m = jnp.max(work, axis=1, keepdims=True)
        idx = jnp.min(jnp.where(work == m, iota, MEM), axis=1, keepdims=True)
        sel = iota == idx
        onehot = jnp.where(sel, jnp.float32(1.0), jnp.float32(0.0))
        v = lax.dot_general(onehot, vals_col, (((1,), (0,)), ((), ())),
                            preferred_element_type=jnp.float32,
                            precision=lax.Precision.HIGHEST)
        top_s.append(m)
        top_v.append(v)
        work = jnp.where(sel, _NEG_INF, work)

    ts_ref[...] = jnp.concatenate(top_s, axis=1)
    rv_ref[...] = jnp.concatenate(top_v, axis=1)
    bm_ref[0, 0, 0] = jnp.max(top_s[0])


@jax.jit
def _retrieve(q2, W, gbb, memory_keys, mv_row):
    B = q2.shape[0]
    grid = B // BT
    ts, rv, bm = pl.pallas_call(
        _fused_kernel,
        grid=(grid,),
        in_specs=[
            pl.BlockSpec((BT, T // 2, 128), lambda i: (i, 0, 0)),
            pl.BlockSpec((5, D_REPR), lambda i: (0, 0)),
            pl.BlockSpec((3, D_REPR), lambda i: (0, 0)),
            pl.BlockSpec((MEM, D_REPR), lambda i: (0, 0)),
            pl.BlockSpec((MEM, 1), lambda i: (0, 0)),
        ],
        out_specs=[
            pl.BlockSpec((BT, TOP_K), lambda i: (i, 0)),
            pl.BlockSpec((BT, TOP_K), lambda i: (i, 0)),
            pl.BlockSpec((1, 1, 1), lambda i: (i, 0, 0),
                         memory_space=pltpu.SMEM),
        ],
        out_shape=[
            jax.ShapeDtypeStruct((B, TOP_K), jnp.float32),
            jax.ShapeDtypeStruct((B, TOP_K), jnp.float32),
            jax.ShapeDtypeStruct((grid, 1, 1), jnp.float32),
        ],
        scratch_shapes=[pltpu.VMEM((MEM, D_REPR), jnp.bfloat16)],
    )(q2, W, gbb, memory_keys, mv_row)
    return ts, rv, bm


def kernel(query, W, b, gamma, beta, memory_keys, memory_values):
    B = query.shape[0]
    q2 = query.reshape(B, T // 2, 2 * 64)
    gbb = jnp.stack([b, gamma, beta], axis=0)
    mv_row = memory_values
    ts, rv, bm = _retrieve(q2, W, gbb, memory_keys, mv_row)
    retrieved_values = rv.reshape(B, TOP_K, 1)
    is_valid = jnp.max(bm) > jnp.float32(THRESH)
    return (retrieved_values, ts, is_valid)


# speedup vs baseline: 1.4559x; 1.4559x over previous
import functools

import jax
import jax.numpy as jnp
from jax import lax
from jax.experimental import pallas as pl
from jax.experimental.pallas import tpu as pltpu

D_REPR = 64
TOP_K = 3
THRESH = 0.95
MEM = 5000
T = 200
BT = 128

_NEG_INF = float("-inf")


def _fused_kernel(q_ref, w_ref, gbb_ref, mk_ref, mv_ref,
                  ts_ref, rv_ref, bm_ref, mkn_ref):
    i = pl.program_id(0)

    @pl.when(i == 0)
    def _():
        mk = mk_ref[...]
        mkn_ref[...] = (mk / jnp.maximum(
            jnp.sqrt(jnp.sum(mk * mk, axis=1, keepdims=True)),
            jnp.float32(1e-12))).astype(jnp.bfloat16)

    xt = jnp.swapaxes(q_ref[...], 1, 2)
    inv_d = jnp.float32(1.0 / D_REPR)
    ae = jnp.sum(xt[:, :64, :], axis=1) * inv_d
    ao = jnp.sum(xt[:, 64:, :], axis=1) * inv_d

    mean = (jnp.sum(ae, axis=1, keepdims=True)
            + jnp.sum(ao, axis=1, keepdims=True)) * jnp.float32(1.0 / T)
    dev2 = (jnp.sum((ae - mean) ** 2, axis=1, keepdims=True)
            + jnp.sum((ao - mean) ** 2, axis=1, keepdims=True))
    std = jnp.sqrt(dev2 * jnp.float32(1.0 / (T - 1)))
    mx = jnp.maximum(jnp.max(ae, axis=1, keepdims=True),
                     jnp.max(ao, axis=1, keepdims=True))
    mn = jnp.minimum(jnp.min(ae, axis=1, keepdims=True),
                     jnp.min(ao, axis=1, keepdims=True))
    trend = ao[:, T // 2 - 1:] - ae[:, :1]
    stats = jnp.concatenate([mean, std, mx, mn, trend], axis=1)

    h = lax.dot_general(stats.astype(jnp.bfloat16),
                        w_ref[...].astype(jnp.bfloat16),
                        (((1,), (0,)), ((), ())),
                        preferred_element_type=jnp.float32)
    h = h + gbb_ref[0:1, :]
    mu = jnp.mean(h, axis=1, keepdims=True)
    var = jnp.mean((h - mu) ** 2, axis=1, keepdims=True)
    h = (h - mu) * lax.rsqrt(var + jnp.float32(1e-5))
    h = h * gbb_ref[1:2, :] + gbb_ref[2:3, :]

    qn = h / jnp.maximum(jnp.sqrt(jnp.sum(h * h, axis=1, keepdims=True)),
                         jnp.float32(1e-12))

    sim = lax.dot_general(qn.astype(jnp.bfloat16), mkn_ref[...],
                          (((1,), (1,)), ((), ())),
                          preferred_element_type=jnp.float32)

    vals = mv_ref[0:1, :]
    iota = lax.broadcasted_iota(jnp.int32, (BT, MEM), 1)
    work = sim
    top_s = []
    top_v = []
    for _ in range(TOP_K):
        m = jnp.max(work, axis=1, keepdims=True)
        idx = jnp.min(jnp.where(work == m, iota, MEM), axis=1, keepdims=True)
        sel = iota == idx
        v = jnp.sum(jnp.where(sel, vals, jnp.float32(0.0)), axis=1,
                    keepdims=True)
        top_s.append(m)
        top_v.append(v)
        work = jnp.where(sel, _NEG_INF, work)

    ts_ref[...] = jnp.concatenate(top_s, axis=1)
    rv_ref[...] = jnp.concatenate(top_v, axis=1)
    bm_ref[0, 0, 0] = jnp.max(top_s[0])


@jax.jit
def _retrieve(q2, W, gbb, memory_keys, mv_row):
    B = q2.shape[0]
    grid = B // BT
    ts, rv, bm = pl.pallas_call(
        _fused_kernel,
        grid=(grid,),
        in_specs=[
            pl.BlockSpec((BT, T // 2, 128), lambda i: (i, 0, 0)),
            pl.BlockSpec((5, D_REPR), lambda i: (0, 0)),
            pl.BlockSpec((3, D_REPR), lambda i: (0, 0)),
            pl.BlockSpec((MEM, D_REPR), lambda i: (0, 0)),
            pl.BlockSpec((1, MEM), lambda i: (0, 0)),
        ],
        out_specs=[
            pl.BlockSpec((BT, TOP_K), lambda i: (i, 0)),
            pl.BlockSpec((BT, TOP_K), lambda i: (i, 0)),
            pl.BlockSpec((1, 1, 1), lambda i: (i, 0, 0),
                         memory_space=pltpu.SMEM),
        ],
        out_shape=[
            jax.ShapeDtypeStruct((B, TOP_K), jnp.float32),
            jax.ShapeDtypeStruct((B, TOP_K), jnp.float32),
            jax.ShapeDtypeStruct((grid, 1, 1), jnp.float32),
        ],
        scratch_shapes=[pltpu.VMEM((MEM, D_REPR), jnp.bfloat16)],
    )(q2, W, gbb, memory_keys, mv_row)
    return ts, rv, bm


def kernel(query, W, b, gamma, beta, memory_keys, memory_values):
    B = query.shape[0]
    q2 = query.reshape(B, T // 2, 2 * 64)
    gbb = jnp.stack([b, gamma, beta], axis=0)
    mv_row = memory_values.reshape(1, MEM)
    ts, rv, bm = _retrieve(q2, W, gbb, memory_keys, mv_row)
    retrieved_values = rv.reshape(B, TOP_K, 1)
    is_valid = jnp.max(bm) > jnp.float32(THRESH)
    return (retrieved_values, ts, is_valid)


# speedup vs baseline: 1.4978x; 1.0288x over previous
import functools

import jax
import jax.numpy as jnp
from jax import lax
from jax.experimental import pallas as pl
from jax.experimental.pallas import tpu as pltpu

D_REPR = 64
TOP_K = 3
THRESH = 0.95
MEM = 5000
T = 200
BT = 256

_NEG_INF = float("-inf")


def _fused_kernel(q_ref, w_ref, gbb_ref, mk_ref, mv_ref,
                  ts_ref, rv_ref, bm_ref, mkn_ref):
    i = pl.program_id(0)

    @pl.when(i == 0)
    def _():
        mk = mk_ref[...]
        mkn_ref[...] = (mk / jnp.maximum(
            jnp.sqrt(jnp.sum(mk * mk, axis=1, keepdims=True)),
            jnp.float32(1e-12))).astype(jnp.bfloat16)

    xt = jnp.swapaxes(q_ref[...], 1, 2)
    inv_d = jnp.float32(1.0 / D_REPR)
    ae = jnp.sum(xt[:, :64, :], axis=1) * inv_d
    ao = jnp.sum(xt[:, 64:, :], axis=1) * inv_d

    mean = (jnp.sum(ae, axis=1, keepdims=True)
            + jnp.sum(ao, axis=1, keepdims=True)) * jnp.float32(1.0 / T)
    dev2 = (jnp.sum((ae - mean) ** 2, axis=1, keepdims=True)
            + jnp.sum((ao - mean) ** 2, axis=1, keepdims=True))
    std = jnp.sqrt(dev2 * jnp.float32(1.0 / (T - 1)))
    mx = jnp.maximum(jnp.max(ae, axis=1, keepdims=True),
                     jnp.max(ao, axis=1, keepdims=True))
    mn = jnp.minimum(jnp.min(ae, axis=1, keepdims=True),
                     jnp.min(ao, axis=1, keepdims=True))
    trend = ao[:, T // 2 - 1:] - ae[:, :1]
    stats = jnp.concatenate([mean, std, mx, mn, trend], axis=1)

    h = lax.dot_general(stats.astype(jnp.bfloat16),
                        w_ref[...].astype(jnp.bfloat16),
                        (((1,), (0,)), ((), ())),
                        preferred_element_type=jnp.float32)
    h = h + gbb_ref[0:1, :]
    mu = jnp.mean(h, axis=1, keepdims=True)
    var = jnp.mean((h - mu) ** 2, axis=1, keepdims=True)
    h = (h - mu) * lax.rsqrt(var + jnp.float32(1e-5))
    h = h * gbb_ref[1:2, :] + gbb_ref[2:3, :]

    qn = h / jnp.maximum(jnp.sqrt(jnp.sum(h * h, axis=1, keepdims=True)),
                         jnp.float32(1e-12))

    sim = lax.dot_general(qn.astype(jnp.bfloat16), mkn_ref[...],
                          (((1,), (1,)), ((), ())),
                          preferred_element_type=jnp.float32)

    vals = mv_ref[0:1, :]
    iota = lax.broadcasted_iota(jnp.int32, (BT, MEM), 1)
    work = sim
    top_s = []
    top_v = []
    for _ in range(TOP_K):
        m = jnp.max(work, axis=1, keepdims=True)
        idx = jnp.min(jnp.where(work == m, iota, MEM), axis=1, keepdims=True)
        sel = iota == idx
        v = jnp.sum(jnp.where(sel, vals, jnp.float32(0.0)), axis=1,
                    keepdims=True)
        top_s.append(m)
        top_v.append(v)
        work = jnp.where(sel, _NEG_INF, work)

    ts_ref[...] = jnp.concatenate(top_s, axis=1)
    rv_ref[...] = jnp.concatenate(top_v, axis=1)
    bm_ref[0, 0, 0] = jnp.max(top_s[0])


@jax.jit
def _retrieve(q2, W, gbb, memory_keys, mv_row):
    B = q2.shape[0]
    grid = B // BT
    ts, rv, bm = pl.pallas_call(
        _fused_kernel,
        grid=(grid,),
        in_specs=[
            pl.BlockSpec((BT, T // 2, 128), lambda i: (i, 0, 0)),
            pl.BlockSpec((5, D_REPR), lambda i: (0, 0)),
            pl.BlockSpec((3, D_REPR), lambda i: (0, 0)),
            pl.BlockSpec((MEM, D_REPR), lambda i: (0, 0)),
            pl.BlockSpec((1, MEM), lambda i: (0, 0)),
        ],
        out_specs=[
            pl.BlockSpec((BT, TOP_K), lambda i: (i, 0)),
            pl.BlockSpec((BT, TOP_K), lambda i: (i, 0)),
            pl.BlockSpec((1, 1, 1), lambda i: (i, 0, 0),
                         memory_space=pltpu.SMEM),
        ],
        out_shape=[
            jax.ShapeDtypeStruct((B, TOP_K), jnp.float32),
            jax.ShapeDtypeStruct((B, TOP_K), jnp.float32),
            jax.ShapeDtypeStruct((grid, 1, 1), jnp.float32),
        ],
        scratch_shapes=[pltpu.VMEM((MEM, D_REPR), jnp.bfloat16)],
    )(q2, W, gbb, memory_keys, mv_row)
    return ts, rv, bm


def kernel(query, W, b, gamma, beta, memory_keys, memory_values):
    B = query.shape[0]
    q2 = query.reshape(B, T // 2, 2 * 64)
    gbb = jnp.stack([b, gamma, beta], axis=0)
    mv_row = memory_values.reshape(1, MEM)
    ts, rv, bm = _retrieve(q2, W, gbb, memory_keys, mv_row)
    retrieved_values = rv.reshape(B, TOP_K, 1)
    is_valid = jnp.max(bm) > jnp.float32(THRESH)
    return (retrieved_values, ts, is_valid)
